# baseline (speedup 1.0000x reference)
import jax
import jax.numpy as jnp
from jax import lax
from jax.experimental import pallas as pl
from jax.experimental.pallas import tpu as pltpu
from jax.experimental.pallas import tpu_sc as plsc

D_MODEL = 64
DPAD = 128
SEQ_LEN = 200
BATCH = 4096
NUM_WORKERS = 32
N_ROWS = BATCH * SEQ_LEN
N_SLABS = 4
SLAB_B = BATCH // N_SLABS
BROWS_PER_W = SLAB_B // NUM_WORKERS
G0 = 128
G1 = SEQ_LEN - G0


def _make_encoding(seq_len: int) -> jax.Array:
    pos = jnp.arange(seq_len, dtype=jnp.float32)[:, None]
    _2i = jnp.arange(0, D_MODEL, 2, dtype=jnp.float32)
    enc = jnp.zeros((seq_len, D_MODEL), dtype=jnp.float32)
    enc = enc.at[:, 0::2].set(jnp.sin(pos / (10000.0 ** (_2i / D_MODEL))))
    enc = enc.at[:, 1::2].set(jnp.cos(pos / (10000.0 ** (_2i / D_MODEL))))
    return enc


def _emb_body(table_hbm, idx_hbm, enc_hbm, out_hbm,
              enc_v, idx0, idx1, rin0, rin1, rout0, rout1,
              sg0, sg1, so0, so1, si0, si1):
    wid = lax.axis_index("s") * 2 + lax.axis_index("c")
    base_b = wid * BROWS_PER_W
    idx = (idx0, idx1)
    rin = (rin0, rin1)
    rout = (rout0, rout1)
    sg = (sg0, sg1)
    so = (so0, so1)
    si = (si0, si1)

    def idx_start(c, b):
        pltpu.make_async_copy(
            idx_hbm.at[pl.ds((base_b + c) * SEQ_LEN, SEQ_LEN)], idx[b], si[b]
        ).start()

    def idx_wait(b):
        pltpu.make_async_copy(
            idx_hbm.at[pl.ds(0, SEQ_LEN)], idx[b], si[b]
        ).wait()

    def gather_start(b):
        pltpu.make_async_copy(
            table_hbm.at[idx[b].at[pl.ds(0, G0)]], rin[b].at[pl.ds(0, G0)],
            sg[b]).start()
        pltpu.make_async_copy(
            table_hbm.at[idx[b].at[pl.ds(G0, G1)]], rin[b].at[pl.ds(G0, G1)],
            sg[b]).start()

    def gather_wait(b):
        pltpu.make_async_copy(
            table_hbm.at[idx[b].at[pl.ds(0, G0)]], rin[b].at[pl.ds(0, G0)],
            sg[b]).wait()
        pltpu.make_async_copy(
            table_hbm.at[idx[b].at[pl.ds(G0, G1)]], rin[b].at[pl.ds(G0, G1)],
            sg[b]).wait()

    def store_start(c, b):
        pltpu.make_async_copy(
            rout[b], out_hbm.at[pl.ds(base_b + c, 1)], so[b]
        ).start()

    def store_wait(b):
        pltpu.make_async_copy(
            rout[b], out_hbm.at[pl.ds(0, 1)], so[b]
        ).wait()

    def add_enc(b):
        src = rin[b]
        dst = rout[b]

        @plsc.parallel_loop(0, SEQ_LEN, 1, unroll=4)
        def _(j):
            for d in range(D_MODEL // 16):
                sl = pl.ds(d * 16, 16)
                dst[0, j, sl] = src[j, sl] + enc_v[j, sl]

    pltpu.sync_copy(enc_hbm, enc_v)

    pltpu.sync_copy(idx_hbm.at[pl.ds(base_b * SEQ_LEN, SEQ_LEN)], idx0)
    gather_start(0)
    idx_start(1, 1)

    def pair_step(g, carry):
        for b in range(2):
            o = 1 - b
            c = 2 * g + b
            if b == 0:
                idx_wait(o)
                gather_start(o)
            else:
                @pl.when(g < BROWS_PER_W // 2 - 1)
                def _():
                    idx_wait(o)
                    gather_start(o)
            gather_wait(b)

            @pl.when(g >= 1)
            def _():
                store_wait(b)

            add_enc(b)
            store_start(c, b)

            @pl.when(g < BROWS_PER_W // 2 - 1)
            def _():
                idx_start(c + 2, b)
        return carry

    lax.fori_loop(0, BROWS_PER_W // 2, pair_step, 0)
    store_wait(0)
    store_wait(1)


@jax.jit
def kernel(x, table):
    seq_len = x.shape[1]
    enc = _make_encoding(seq_len)
    table_p = jnp.pad(table, ((0, 0), (0, DPAD - D_MODEL)))

    mesh = plsc.VectorSubcoreMesh(core_axis_name="c", subcore_axis_name="s")
    run = pl.kernel(
        _emb_body,
        out_type=jax.ShapeDtypeStruct((SLAB_B, SEQ_LEN, D_MODEL), jnp.float32),
        mesh=mesh,
        scratch_types=[
            pltpu.VMEM((SEQ_LEN, D_MODEL), jnp.float32),
            pltpu.VMEM((SEQ_LEN,), jnp.int32),
            pltpu.VMEM((SEQ_LEN,), jnp.int32),
            pltpu.VMEM((SEQ_LEN, DPAD), jnp.float32),
            pltpu.VMEM((SEQ_LEN, DPAD), jnp.float32),
            pltpu.VMEM((1, SEQ_LEN, D_MODEL), jnp.float32),
            pltpu.VMEM((1, SEQ_LEN, D_MODEL), jnp.float32),
            pltpu.SemaphoreType.DMA,
            pltpu.SemaphoreType.DMA,
            pltpu.SemaphoreType.DMA,
            pltpu.SemaphoreType.DMA,
            pltpu.SemaphoreType.DMA,
            pltpu.SemaphoreType.DMA,
        ],
        compiler_params=pltpu.CompilerParams(use_tc_tiling_on_sc=True),
    )
    slabs = [
        run(table_p, x[i * SLAB_B:(i + 1) * SLAB_B].reshape(-1), enc)
        for i in range(N_SLABS)
    ]
    return jnp.concatenate(slabs, axis=0)


# speedup vs baseline: 1.1776x; 1.1776x over previous
import jax
import jax.numpy as jnp
from jax import lax
from jax.experimental import pallas as pl
from jax.experimental.pallas import tpu as pltpu
from jax.experimental.pallas import tpu_sc as plsc

D_MODEL = 64
DPAD = 128
SEQ_LEN = 200
BATCH = 4096
NUM_WORKERS = 32
BW = BATCH // NUM_WORKERS


def _make_encoding(seq_len: int) -> jax.Array:
    pos = jnp.arange(seq_len, dtype=jnp.float32)[:, None]
    _2i = jnp.arange(0, D_MODEL, 2, dtype=jnp.float32)
    enc = jnp.zeros((seq_len, D_MODEL), dtype=jnp.float32)
    enc = enc.at[:, 0::2].set(jnp.sin(pos / (10000.0 ** (_2i / D_MODEL))))
    enc = enc.at[:, 1::2].set(jnp.cos(pos / (10000.0 ** (_2i / D_MODEL))))
    return enc


def _emb_body(table_hbm, xt_hbm, enc_hbm, out_hbm,
              idx_v, enc_v, rin0, rin1, rtr0, rtr1,
              sg0, sg1, so0, so1):
    wid = lax.axis_index("s") * 2 + lax.axis_index("c")
    b0 = wid * BW
    rin = (rin0, rin1)
    rtr = (rtr0, rtr1)
    sg = (sg0, sg1)
    so = (so0, so1)

    def gather_start(s, b):
        pltpu.make_async_copy(
            table_hbm.at[idx_v.at[s]], rin[b], sg[b]).start()

    def gather_wait(b):
        pltpu.make_async_copy(
            table_hbm.at[idx_v.at[0]], rin[b], sg[b]).wait()

    def store_start(s, b):
        pltpu.make_async_copy(
            rtr[b], out_hbm.at[pl.ds(s, 1), :, pl.ds(b0, BW)], so[b]
        ).start()

    def store_wait(b):
        pltpu.make_async_copy(
            rtr[b], out_hbm.at[pl.ds(0, 1), :, pl.ds(b0, BW)], so[b]
        ).wait()

    iota16 = lax.broadcasted_iota(jnp.int32, (16,), 0)
    bg = [iota16 + 16 * g for g in range(BW // 16)]

    def transpose_add(s, b):
        src = rin[b]
        dst = rtr[b]
        s16 = jnp.full((16,), s, jnp.int32)

        @plsc.parallel_loop(0, D_MODEL, 1, unroll=2)
        def _(d):
            d16 = jnp.full((16,), d, jnp.int32)
            e = plsc.load_gather(enc_v, [s16, d16])
            for g in range(BW // 16):
                v = plsc.load_gather(src, [bg[g], d16])
                dst[0, d, pl.ds(16 * g, 16)] = v + e

    pltpu.sync_copy(xt_hbm.at[:, pl.ds(b0, BW)], idx_v)
    pltpu.sync_copy(enc_hbm, enc_v)
    gather_start(0, 0)

    def pair_step(g, carry):
        for b in range(2):
            o = 1 - b
            s = 2 * g + b
            if b == 0:
                gather_start(s + 1, o)
            else:
                @pl.when(g < SEQ_LEN // 2 - 1)
                def _():
                    gather_start(s + 1, o)
            gather_wait(b)

            @pl.when(g >= 1)
            def _():
                store_wait(b)

            transpose_add(s, b)
            store_start(s, b)
        return carry

    lax.fori_loop(0, SEQ_LEN // 2, pair_step, 0)
    store_wait(0)
    store_wait(1)


@jax.jit
def kernel(x, table):
    seq_len = x.shape[1]
    enc = jnp.pad(_make_encoding(seq_len), ((0, 0), (0, DPAD - D_MODEL)))
    table_p = jnp.pad(table, ((0, 0), (0, DPAD - D_MODEL)))
    xt = x.T

    mesh = plsc.VectorSubcoreMesh(core_axis_name="c", subcore_axis_name="s")
    run = pl.kernel(
        _emb_body,
        out_type=jax.ShapeDtypeStruct((SEQ_LEN, D_MODEL, BATCH), jnp.float32),
        mesh=mesh,
        scratch_types=[
            pltpu.VMEM((SEQ_LEN, BW), jnp.int32),
            pltpu.VMEM((SEQ_LEN, DPAD), jnp.float32),
            pltpu.VMEM((BW, DPAD), jnp.float32),
            pltpu.VMEM((BW, DPAD), jnp.float32),
            pltpu.VMEM((1, D_MODEL, BW), jnp.float32),
            pltpu.VMEM((1, D_MODEL, BW), jnp.float32),
            pltpu.SemaphoreType.DMA,
            pltpu.SemaphoreType.DMA,
            pltpu.SemaphoreType.DMA,
            pltpu.SemaphoreType.DMA,
        ],
        compiler_params=pltpu.CompilerParams(
            use_tc_tiling_on_sc=True, needs_layout_passes=False),
    )
    out = run(table_p, xt, enc)
    return out.transpose(2, 0, 1)


# speedup vs baseline: 2.7326x; 2.3206x over previous
import jax
import jax.numpy as jnp
from jax import lax
from jax.experimental import pallas as pl
from jax.experimental.pallas import tpu as pltpu
from jax.experimental.pallas import tpu_sc as plsc

D_MODEL = 64
DPAD = 128
SEQ_LEN = 200
BATCH = 4096
NUM_WORKERS = 32
BW = BATCH // NUM_WORKERS


def _make_encoding(seq_len: int) -> jax.Array:
    pos = jnp.arange(seq_len, dtype=jnp.float32)[:, None]
    _2i = jnp.arange(0, D_MODEL, 2, dtype=jnp.float32)
    enc = jnp.zeros((seq_len, D_MODEL), dtype=jnp.float32)
    enc = enc.at[:, 0::2].set(jnp.sin(pos / (10000.0 ** (_2i / D_MODEL))))
    enc = enc.at[:, 1::2].set(jnp.cos(pos / (10000.0 ** (_2i / D_MODEL))))
    return enc


def _emb_body(table_hbm, xt_hbm, enc_hbm, out_hbm,
              idx_v, enc_v, rin0, rin1, rtr0, rtr1,
              sg0, sg1, so0, so1):
    wid = lax.axis_index("s") * 2 + lax.axis_index("c")
    b0 = wid * BW
    rin = (rin0, rin1)
    rtr = (rtr0, rtr1)
    sg = (sg0, sg1)
    so = (so0, so1)

    def gather_start(s, b):
        pltpu.make_async_copy(
            table_hbm.at[idx_v.at[s]], rin[b], sg[b]).start()

    def gather_wait(b):
        pltpu.make_async_copy(
            table_hbm.at[idx_v.at[0]], rin[b], sg[b]).wait()

    def store_start(s, b):
        pltpu.make_async_copy(
            rtr[b], out_hbm.at[pl.ds(s, 1), :, pl.ds(b0, BW)], so[b]
        ).start()

    def store_wait(b):
        pltpu.make_async_copy(
            rtr[b], out_hbm.at[pl.ds(0, 1), :, pl.ds(b0, BW)], so[b]
        ).wait()

    iota16 = lax.broadcasted_iota(jnp.int32, (16,), 0)
    rot = [lax.bitwise_and(iota16 + g, jnp.int32(15)) for g in range(16)]
    n_blk = (D_MODEL // 16) * (BW // 16)

    def transpose_add(s, b):
        src = rin[b]
        dst = rtr[b].at[0]
        s16 = jnp.full((16,), s, jnp.int32)

        @plsc.parallel_loop(0, n_blk, 1, unroll=2)
        def _(blk):
            d0 = lax.shift_left(lax.shift_right_logical(blk, 3), 4)
            bb = lax.shift_left(lax.bitwise_and(blk, jnp.int32(7)), 4)
            bvec = iota16 + bb
            for g in range(16):
                dvec = rot[g] + d0
                e = plsc.load_gather(enc_v, [s16, dvec])
                v = plsc.load_gather(src, [bvec, dvec])
                plsc.store_scatter(dst, [dvec, bvec], v + e)

    pltpu.sync_copy(xt_hbm.at[:, pl.ds(b0, BW)], idx_v)
    pltpu.sync_copy(enc_hbm, enc_v)
    gather_start(0, 0)

    def pair_step(g, carry):
        for b in range(2):
            o = 1 - b
            s = 2 * g + b
            if b == 0:
                gather_start(s + 1, o)
            else:
                @pl.when(g < SEQ_LEN // 2 - 1)
                def _():
                    gather_start(s + 1, o)
            gather_wait(b)

            @pl.when(g >= 1)
            def _():
                store_wait(b)

            transpose_add(s, b)
            store_start(s, b)
        return carry

    lax.fori_loop(0, SEQ_LEN // 2, pair_step, 0)
    store_wait(0)
    store_wait(1)


@jax.jit
def kernel(x, table):
    seq_len = x.shape[1]
    enc = jnp.pad(_make_encoding(seq_len), ((0, 0), (0, DPAD - D_MODEL)))
    table_p = jnp.pad(table, ((0, 0), (0, DPAD - D_MODEL)))
    xt = x.T

    mesh = plsc.VectorSubcoreMesh(core_axis_name="c", subcore_axis_name="s")
    run = pl.kernel(
        _emb_body,
        out_type=jax.ShapeDtypeStruct((SEQ_LEN, D_MODEL, BATCH), jnp.float32),
        mesh=mesh,
        scratch_types=[
            pltpu.VMEM((SEQ_LEN, BW), jnp.int32),
            pltpu.VMEM((SEQ_LEN, DPAD), jnp.float32),
            pltpu.VMEM((BW, DPAD), jnp.float32),
            pltpu.VMEM((BW, DPAD), jnp.float32),
            pltpu.VMEM((1, D_MODEL, BW), jnp.float32),
            pltpu.VMEM((1, D_MODEL, BW), jnp.float32),
            pltpu.SemaphoreType.DMA,
            pltpu.SemaphoreType.DMA,
            pltpu.SemaphoreType.DMA,
            pltpu.SemaphoreType.DMA,
        ],
        compiler_params=pltpu.CompilerParams(
            use_tc_tiling_on_sc=True, needs_layout_passes=False),
    )
    out = run(table_p, xt, enc)
    return out.transpose(2, 0, 1)


# speedup vs baseline: 2.7800x; 1.0173x over previous
import jax
import jax.numpy as jnp
from jax import lax
from jax.experimental import pallas as pl
from jax.experimental.pallas import tpu as pltpu
from jax.experimental.pallas import tpu_sc as plsc

D_MODEL = 64
DPAD = 128
SEQ_LEN = 200
BATCH = 4096
NUM_WORKERS = 32
BW = BATCH // NUM_WORKERS


def _make_encoding(seq_len: int) -> jax.Array:
    pos = jnp.arange(seq_len, dtype=jnp.float32)[:, None]
    _2i = jnp.arange(0, D_MODEL, 2, dtype=jnp.float32)
    enc = jnp.zeros((seq_len, D_MODEL), dtype=jnp.float32)
    enc = enc.at[:, 0::2].set(jnp.sin(pos / (10000.0 ** (_2i / D_MODEL))))
    enc = enc.at[:, 1::2].set(jnp.cos(pos / (10000.0 ** (_2i / D_MODEL))))
    return enc


def _emb_body(table_hbm, xt_hbm, enc_hbm, out_hbm,
              idx_v, enc_v, rin0, rin1, rtr0, rtr1,
              sg0, sg1, so0, so1):
    wid = lax.axis_index("s") * 2 + lax.axis_index("c")
    b0 = wid * BW
    rin = (rin0, rin1)
    rtr = (rtr0, rtr1)
    sg = (sg0, sg1)
    so = (so0, so1)

    def gather_start(s, b):
        pltpu.make_async_copy(
            table_hbm.at[idx_v.at[s]], rin[b], sg[b]).start()

    def gather_wait(b):
        pltpu.make_async_copy(
            table_hbm.at[idx_v.at[0]], rin[b], sg[b]).wait()

    def store_start(s, b):
        pltpu.make_async_copy(
            rtr[b], out_hbm.at[pl.ds(s, 1), :, pl.ds(b0, BW)], so[b]
        ).start()

    def store_wait(b):
        pltpu.make_async_copy(
            rtr[b], out_hbm.at[pl.ds(0, 1), :, pl.ds(b0, BW)], so[b]
        ).wait()

    iota16 = lax.broadcasted_iota(jnp.int32, (16,), 0)
    rot = [lax.bitwise_and(iota16 + g, jnp.int32(15)) for g in range(16)]
    n_blk = (D_MODEL // 16) * (BW // 16)

    def transpose_add(s, b):
        src = rin[b]
        dst = rtr[b].at[0]
        s16 = jnp.full((16,), s, jnp.int32)

        @plsc.parallel_loop(0, n_blk, 1, unroll=4)
        def _(blk):
            d0 = lax.shift_left(lax.shift_right_logical(blk, 3), 4)
            bb = lax.shift_left(lax.bitwise_and(blk, jnp.int32(7)), 4)
            bvec = iota16 + bb
            for g in range(16):
                dvec = rot[g] + d0
                e = plsc.load_gather(enc_v, [s16, dvec])
                v = plsc.load_gather(src, [bvec, dvec])
                plsc.store_scatter(dst, [dvec, bvec], v + e)

    pltpu.sync_copy(xt_hbm.at[:, pl.ds(b0, BW)], idx_v)
    pltpu.sync_copy(enc_hbm, enc_v)
    gather_start(0, 0)

    def pair_step(g, carry):
        for b in range(2):
            o = 1 - b
            s = 2 * g + b
            if b == 0:
                gather_start(s + 1, o)
            else:
                @pl.when(g < SEQ_LEN // 2 - 1)
                def _():
                    gather_start(s + 1, o)
            gather_wait(b)

            @pl.when(g >= 1)
            def _():
                store_wait(b)

            transpose_add(s, b)
            store_start(s, b)
        return carry

    lax.fori_loop(0, SEQ_LEN // 2, pair_step, 0)
    store_wait(0)
    store_wait(1)


@jax.jit
def kernel(x, table):
    seq_len = x.shape[1]
    enc = jnp.pad(_make_encoding(seq_len), ((0, 0), (0, DPAD - D_MODEL)))
    table_p = jnp.pad(table, ((0, 0), (0, DPAD - D_MODEL)))
    xt = x.T

    mesh = plsc.VectorSubcoreMesh(core_axis_name="c", subcore_axis_name="s")
    run = pl.kernel(
        _emb_body,
        out_type=jax.ShapeDtypeStruct((SEQ_LEN, D_MODEL, BATCH), jnp.float32),
        mesh=mesh,
        scratch_types=[
            pltpu.VMEM((SEQ_LEN, BW), jnp.int32),
            pltpu.VMEM((SEQ_LEN, DPAD), jnp.float32),
            pltpu.VMEM((BW, DPAD), jnp.float32),
            pltpu.VMEM((BW, DPAD), jnp.float32),
            pltpu.VMEM((1, D_MODEL, BW), jnp.float32),
            pltpu.VMEM((1, D_MODEL, BW), jnp.float32),
            pltpu.SemaphoreType.DMA,
            pltpu.SemaphoreType.DMA,
            pltpu.SemaphoreType.DMA,
            pltpu.SemaphoreType.DMA,
        ],
        compiler_params=pltpu.CompilerParams(
            use_tc_tiling_on_sc=True, needs_layout_passes=False),
    )
    out = run(table_p, xt, enc)
    return out.transpose(2, 0, 1)
